# baseline (speedup 1.0000x reference)
import functools

import jax
import jax.numpy as jnp
from jax import lax
from jax.experimental import pallas as pl
from jax.experimental.pallas import tpu as pltpu
from jax.experimental.pallas import tpu_sc as plsc

B = 4096
T = 200
D = 64
VOCAB = 1000000
VPAD = 123 * 8192
NM = 2
NBUF = 4
NPAD = 128


@functools.lru_cache(maxsize=None)
def _build_sc(num_cores: int, num_subcores: int):
    nw = num_cores * num_subcores
    spw = B // nw

    mesh = plsc.VectorSubcoreMesh(core_axis_name="c", subcore_axis_name="s",
                                  num_cores=num_cores,
                                  num_subcores=num_subcores)

    @functools.partial(
        pl.kernel,
        out_type=jax.ShapeDtypeStruct((B * 32,), jnp.float32),
        mesh=mesh,
        compiler_params=pltpu.CompilerParams(
            needs_layout_passes=False, use_tc_tiling_on_sc=False),
        scratch_types=[
            pltpu.VMEM((spw * T,), jnp.int32),
            pltpu.VMEM_SHARED((VPAD,), jnp.float32),
            *[pltpu.VMEM((208,), jnp.float32) for _ in range(NBUF)],
            pltpu.VMEM((spw * 32,), jnp.float32),
            *[pltpu.SemaphoreType.DMA for _ in range(NBUF)],
        ],
    )
    def sc_kernel(ids_hbm, ptab_hbm, out_hbm, ids_v, shared, *rest):
        bufs = rest[:NBUF]
        out_v = rest[NBUF]
        sems = rest[NBUF + 1:]

        sid = lax.axis_index("s")
        wid = sid * num_cores + lax.axis_index("c")
        seq0 = wid * spw

        chunk = VPAD // num_subcores
        r0 = sid * chunk
        pltpu.sync_copy(ptab_hbm.at[pl.ds(r0, chunk)],
                        shared.at[pl.ds(r0, chunk)])

        pltpu.sync_copy(ids_hbm.at[pl.ds(seq0 * T, spw * T)], ids_v)

        zeros = jnp.zeros((16,), jnp.float32)
        for p in range(NBUF):
            bufs[p][pl.ds(192, 16)] = zeros
        plsc.subcore_barrier()

        def issue(s, buf, sem):
            pltpu.async_copy(shared.at[ids_v.at[pl.ds(s * T, T)]],
                             buf.at[pl.ds(0, T)], sem)

        def wait(buf, sem):
            pltpu.make_async_copy(shared.at[ids_v.at[pl.ds(0, T)]],
                                  buf.at[pl.ds(0, T)], sem).wait()

        def process(s, buf):
            def acc_body(k, carry):
                a0, a1 = carry
                w = buf[pl.ds(k * 16, 16)]
                lo, hi = plsc.unpack(plsc.bitcast(w, jnp.bfloat16),
                                     format=plsc.PackFormat.INTERLEAVED)
                return (a0 + lo, a1 + hi)

            a0, a1 = lax.fori_loop(0, 13, acc_body, (zeros, zeros),
                                   unroll=13)
            out_v[pl.ds(s * 32, 16)] = a0
            out_v[pl.ds(s * 32 + 16, 16)] = a1

        for p in range(NBUF):
            issue(p, bufs[p], sems[p])

        def outer(i, c):
            s0 = i * NBUF
            for p in range(NBUF):
                s = s0 + p
                wait(bufs[p], sems[p])
                process(s, bufs[p])

                @pl.when(s + NBUF < spw)
                def _():
                    issue(s + NBUF, bufs[p], sems[p])
            return c

        lax.fori_loop(0, spw // NBUF, outer, 0)
        pltpu.sync_copy(out_v, out_hbm.at[pl.ds(seq0 * 32, spw * 32)])

    return sc_kernel


def _tc_project(emb, wt2T):
    def proj_kernel(x_ref, w_ref, o_ref):
        o_ref[...] = jnp.dot(x_ref[...], w_ref[...],
                             preferred_element_type=jnp.float32).T

    bm = 8192
    grid = VPAD // bm
    return pl.pallas_call(
        proj_kernel,
        out_shape=jax.ShapeDtypeStruct((NM, VPAD), jnp.float32),
        grid=(grid,),
        in_specs=[
            pl.BlockSpec((bm, D), lambda i: (i, 0)),
            pl.BlockSpec((D, NM), lambda i: (0, 0)),
        ],
        out_specs=pl.BlockSpec((NM, bm), lambda i: (0, i)),
    )(emb, wt2T)


def _tc_fold(acc32, fold, bp):
    def fold_kernel(x_ref, f_ref, b_ref, o_ref):
        o_ref[...] = (
            jnp.dot(x_ref[...], f_ref[...],
                    preferred_element_type=jnp.float32) + b_ref[...])

    grid = 8
    bb = B // grid
    return pl.pallas_call(
        fold_kernel,
        out_shape=jax.ShapeDtypeStruct((B, NPAD), jnp.float32),
        grid=(grid,),
        in_specs=[
            pl.BlockSpec((bb, 32), lambda i: (i, 0)),
            pl.BlockSpec((32, NPAD), lambda i: (0, 0)),
            pl.BlockSpec((1, NPAD), lambda i: (0, 0)),
        ],
        out_specs=pl.BlockSpec((bb, NPAD), lambda i: (i, 0)),
    )(acc32, fold, bp)


def kernel(input_ids, embedding, W, b):
    info = plsc.get_sparse_core_info()
    sc_kernel = _build_sc(info.num_cores, info.num_subcores)

    p32 = _tc_project(embedding, W * (1.0 / T))
    lo = lax.bitcast_convert_type(p32[0].astype(jnp.bfloat16),
                                  jnp.uint16).astype(jnp.uint32)
    hi = lax.bitcast_convert_type(p32[1].astype(jnp.bfloat16),
                                  jnp.uint16).astype(jnp.uint32)
    ptab = lax.bitcast_convert_type(lo | (hi << 16), jnp.float32)

    ids_flat = input_ids.reshape(-1).astype(jnp.int32)
    acc32 = sc_kernel(ids_flat, ptab).reshape(B, 32)

    lane = jnp.arange(32) // 16
    fold = (lane[:, None] == jnp.arange(NPAD)[None, :]).astype(jnp.float32)
    bp = jnp.pad(b, (0, NPAD - NM)).reshape(1, NPAD)
    return _tc_fold(acc32, fold, bp)[:, :NM]


# speedup vs baseline: 1.1157x; 1.1157x over previous
import functools

import jax
import jax.numpy as jnp
from jax import lax
from jax.experimental import pallas as pl
from jax.experimental.pallas import tpu as pltpu
from jax.experimental.pallas import tpu_sc as plsc

B = 4096
T = 200
D = 64
VOCAB = 1000000
VPAD = 62 * 16384
NM = 2
NBUF = 4
NPAD = 128


@functools.lru_cache(maxsize=None)
def _build_sc(num_cores: int, num_subcores: int):
    nw = num_cores * num_subcores
    spw = B // nw

    mesh = plsc.VectorSubcoreMesh(core_axis_name="c", subcore_axis_name="s",
                                  num_cores=num_cores,
                                  num_subcores=num_subcores)

    @functools.partial(
        pl.kernel,
        out_type=jax.ShapeDtypeStruct((B * 32,), jnp.float32),
        mesh=mesh,
        compiler_params=pltpu.CompilerParams(
            needs_layout_passes=False, use_tc_tiling_on_sc=False),
        scratch_types=[
            pltpu.VMEM((spw * T,), jnp.int32),
            pltpu.VMEM_SHARED((VPAD,), jnp.float32),
            *[pltpu.VMEM((208,), jnp.float32) for _ in range(NBUF)],
            pltpu.VMEM((spw * 32,), jnp.float32),
            *[pltpu.SemaphoreType.DMA for _ in range(NBUF)],
        ],
    )
    def sc_kernel(ids_hbm, ptab_hbm, out_hbm, ids_v, shared, *rest):
        bufs = rest[:NBUF]
        out_v = rest[NBUF]
        sems = rest[NBUF + 1:]

        sid = lax.axis_index("s")
        wid = sid * num_cores + lax.axis_index("c")
        seq0 = wid * spw

        chunk = VPAD // num_subcores
        r0 = sid * chunk
        pltpu.sync_copy(ptab_hbm.at[pl.ds(r0, chunk)],
                        shared.at[pl.ds(r0, chunk)])

        pltpu.sync_copy(ids_hbm.at[pl.ds(seq0 * T, spw * T)], ids_v)

        zeros = jnp.zeros((16,), jnp.float32)
        for p in range(NBUF):
            bufs[p][pl.ds(192, 16)] = zeros
        plsc.subcore_barrier()

        def issue(s, buf, sem):
            pltpu.async_copy(shared.at[ids_v.at[pl.ds(s * T, T)]],
                             buf.at[pl.ds(0, T)], sem)

        def wait(buf, sem):
            pltpu.make_async_copy(shared.at[ids_v.at[pl.ds(0, T)]],
                                  buf.at[pl.ds(0, T)], sem).wait()

        def process(s, buf):
            def acc_body(k, carry):
                a0, a1 = carry
                w = buf[pl.ds(k * 16, 16)]
                lo, hi = plsc.unpack(plsc.bitcast(w, jnp.bfloat16),
                                     format=plsc.PackFormat.INTERLEAVED)
                return (a0 + lo, a1 + hi)

            a0, a1 = lax.fori_loop(0, 13, acc_body, (zeros, zeros),
                                   unroll=13)
            out_v[pl.ds(s * 32, 16)] = a0
            out_v[pl.ds(s * 32 + 16, 16)] = a1

        for p in range(NBUF):
            issue(p, bufs[p], sems[p])

        def outer(i, c):
            s0 = i * NBUF
            for p in range(NBUF):
                s = s0 + p
                wait(bufs[p], sems[p])
                process(s, bufs[p])

                @pl.when(s + NBUF < spw)
                def _():
                    issue(s + NBUF, bufs[p], sems[p])
            return c

        lax.fori_loop(0, spw // NBUF, outer, 0)
        pltpu.sync_copy(out_v, out_hbm.at[pl.ds(seq0 * 32, spw * 32)])

    return sc_kernel


def _tc_project(emb, wt2T):
    def proj_kernel(x_ref, w_ref, o_ref):
        o_ref[...] = lax.dot_general(
            w_ref[...], x_ref[...],
            (((1,), (1,)), ((), ())),
            preferred_element_type=jnp.float32)

    bm = 16384
    grid = VPAD // bm
    return pl.pallas_call(
        proj_kernel,
        out_shape=jax.ShapeDtypeStruct((NM, VPAD), jnp.float32),
        grid=(grid,),
        in_specs=[
            pl.BlockSpec((bm, D), lambda i: (i, 0)),
            pl.BlockSpec((NM, D), lambda i: (0, 0)),
        ],
        out_specs=pl.BlockSpec((NM, bm), lambda i: (0, i)),
    )(emb, wt2T)


def _tc_fold(acc32, fold, bp):
    def fold_kernel(x_ref, f_ref, b_ref, o_ref):
        o_ref[...] = (
            jnp.dot(x_ref[...], f_ref[...],
                    preferred_element_type=jnp.float32) + b_ref[...])

    grid = 8
    bb = B // grid
    return pl.pallas_call(
        fold_kernel,
        out_shape=jax.ShapeDtypeStruct((B, NPAD), jnp.float32),
        grid=(grid,),
        in_specs=[
            pl.BlockSpec((bb, 32), lambda i: (i, 0)),
            pl.BlockSpec((32, NPAD), lambda i: (0, 0)),
            pl.BlockSpec((1, NPAD), lambda i: (0, 0)),
        ],
        out_specs=pl.BlockSpec((bb, NPAD), lambda i: (i, 0)),
    )(acc32, fold, bp)


def kernel(input_ids, embedding, W, b):
    info = plsc.get_sparse_core_info()
    sc_kernel = _build_sc(info.num_cores, info.num_subcores)

    p32 = _tc_project(embedding, (W * (1.0 / T)).T)
    lo = lax.bitcast_convert_type(p32[0].astype(jnp.bfloat16),
                                  jnp.uint16).astype(jnp.uint32)
    hi = lax.bitcast_convert_type(p32[1].astype(jnp.bfloat16),
                                  jnp.uint16).astype(jnp.uint32)
    ptab = lax.bitcast_convert_type(lo | (hi << 16), jnp.float32)

    ids_flat = input_ids.reshape(-1).astype(jnp.int32)
    acc32 = sc_kernel(ids_flat, ptab).reshape(B, 32)

    lane = jnp.arange(32) // 16
    fold = (lane[:, None] == jnp.arange(NPAD)[None, :]).astype(jnp.float32)
    bp = jnp.pad(b, (0, NPAD - NM)).reshape(1, NPAD)
    return _tc_fold(acc32, fold, bp)[:, :NM]
